# baseline (speedup 1.0000x reference)
import jax
import jax.numpy as jnp
from jax import lax
from jax.experimental import pallas as pl
from jax.experimental.pallas import tpu as pltpu
from jax.experimental.pallas import tpu_sc as plsc

BLOCK = 16
VOCAB = 1024
D = 768

_info = plsc.get_sparse_core_info()
NC, NS, L = _info.num_cores, _info.num_subcores, _info.num_lanes
NW = NC * NS

CH = 16
RING = 8
LOOK = 6


def _sc_body(tokens_hbm, token_w_hbm, seq_hbm,
             tok_v, tokrows0, tokrows1, tokrows2, tokrows3,
             tokrows4, tokrows5, tokrows6, tokrows7,
             gt0, gt1, gt2, gt3, gt4, gt5, gt6, gt7,
             ot0, ot1, ot2, ot3, ot4, ot5, ot6, ot7):
    tokrows = (tokrows0, tokrows1, tokrows2, tokrows3,
               tokrows4, tokrows5, tokrows6, tokrows7)
    gt = (gt0, gt1, gt2, gt3, gt4, gt5, gt6, gt7)
    ot = (ot0, ot1, ot2, ot3, ot4, ot5, ot6, ot7)

    batch, lseq = tokens_hbm.shape
    blocks_total = batch * lseq // BLOCK
    blk_per_w = blocks_total // NW
    tok_per_w = blk_per_w * BLOCK
    n_ch = tok_per_w // CH
    npb = lseq // BLOCK
    spb = 2 * npb + lseq
    w_per_b = NW // batch

    wid = lax.axis_index("s") * NC + lax.axis_index("c")
    b = wid // w_per_b
    col0 = (wid - b * w_per_b) * tok_per_w

    pltpu.sync_copy(tokens_hbm.at[b, pl.ds(col0, tok_per_w)], tok_v)

    def issue(c, p):
        idx = tok_v.at[pl.ds(c * CH, CH)]
        pltpu.async_copy(token_w_hbm.at[idx], tokrows[p], gt[p])

    def wait_gather(p):
        pltpu.make_async_copy(token_w_hbm.at[pl.ds(0, CH)],
                              tokrows[p], gt[p]).wait()

    def drain_out(p):
        pltpu.make_async_copy(tokrows[p], seq_hbm.at[pl.ds(0, CH)],
                              ot[p]).wait()

    for c0 in range(LOOK):
        issue(c0, c0)

    def outer(kk, _):
        for i in range(RING):
            c = kk * RING + i
            nc = c + LOOK
            nbuf = (i + LOOK) % RING

            @pl.when(jnp.logical_and(c >= RING - LOOK, nc < n_ch))
            def _():
                drain_out(nbuf)

            @pl.when(nc < n_ch)
            def _():
                issue(nc, nbuf)

            wait_gather(i)
            row0 = b * spb + 2 * npb + col0 + c * CH
            pltpu.async_copy(tokrows[i], seq_hbm.at[pl.ds(row0, CH)], ot[i])
        return 0

    lax.fori_loop(0, n_ch // RING, outer, 0)
    for i in range(RING):
        drain_out(i)


def _tc_body(tokens_ref, cat_w_ref, num_w_ref,
             hist_ref, catnum_ref, cat_ids_ref):
    m = tokens_ref.shape[0]
    npb = 256
    batch = m // npb
    iota_v = lax.broadcasted_iota(jnp.int32, (m, VOCAB), 1)

    col0 = tokens_ref[:, 0:1]
    cat_ids_ref[...] = col0.reshape(batch, npb)

    h = jnp.zeros((m, VOCAB), jnp.float32)
    for r in range(BLOCK):
        col = tokens_ref[:, r:r + 1]
        h = h + (col == iota_v).astype(jnp.float32)
    hist_ref[...] = h

    onehot = (col0 == iota_v).astype(jnp.float32)
    cat = jnp.dot(onehot, cat_w_ref[...], preferred_element_type=jnp.float32)
    num = jnp.dot(h, num_w_ref[...], preferred_element_type=jnp.float32)
    for b in range(batch):
        catnum_ref[b * 2 * npb:b * 2 * npb + npb, :] = (
            cat[b * npb:(b + 1) * npb, :])
        catnum_ref[b * 2 * npb + npb:(b + 1) * 2 * npb, :] = (
            num[b * npb:(b + 1) * npb, :])


def kernel(tokens, cat_W, num_W, token_W):
    B, Lseq = tokens.shape
    n_blocks = Lseq // BLOCK
    seq_rows = 2 * n_blocks + Lseq

    mesh = plsc.VectorSubcoreMesh(core_axis_name="c", subcore_axis_name="s")
    sc = pl.kernel(
        _sc_body,
        out_type=jax.ShapeDtypeStruct((B * seq_rows, D), jnp.float32),
        mesh=mesh,
        compiler_params=pltpu.CompilerParams(needs_layout_passes=False),
        scratch_types=[
            pltpu.VMEM((Lseq * B // NW,), jnp.int32),
        ] + [pltpu.VMEM((CH, D), jnp.float32)] * RING
          + [pltpu.SemaphoreType.DMA] * (2 * RING),
    )

    tc = pl.pallas_call(
        _tc_body,
        out_shape=[
            jax.ShapeDtypeStruct((B * n_blocks, VOCAB), jnp.float32),
            jax.ShapeDtypeStruct((B * 2 * n_blocks, D), jnp.float32),
            jax.ShapeDtypeStruct((B, n_blocks), jnp.int32),
        ],
    )

    seq_flat = sc(tokens, token_W)
    hist_flat, catnum, cat_ids = tc(
        tokens.reshape(B * n_blocks, BLOCK), cat_W, num_W)

    new_seq = seq_flat.reshape(B, seq_rows, D)
    new_seq = lax.dynamic_update_slice(
        new_seq, catnum.reshape(B, 2 * n_blocks, D), (0, 0, 0))
    hist = hist_flat.reshape(B, n_blocks, VOCAB)
    return (new_seq, cat_ids, hist)


# speedup vs baseline: 1.0017x; 1.0017x over previous
import jax
import jax.numpy as jnp
from jax import lax
from jax.experimental import pallas as pl
from jax.experimental.pallas import tpu as pltpu
from jax.experimental.pallas import tpu_sc as plsc

BLOCK = 16
VOCAB = 1024
D = 768

_info = plsc.get_sparse_core_info()
NC, NS, L = _info.num_cores, _info.num_subcores, _info.num_lanes
NW = NC * NS

CH = 16
RING = 8
LOOK = 4


def _sc_body(tokens_hbm, token_w_hbm, seq_hbm,
             tok_v, tokrows0, tokrows1, tokrows2, tokrows3,
             tokrows4, tokrows5, tokrows6, tokrows7,
             gt0, gt1, gt2, gt3, gt4, gt5, gt6, gt7,
             ot0, ot1, ot2, ot3, ot4, ot5, ot6, ot7):
    tokrows = (tokrows0, tokrows1, tokrows2, tokrows3,
               tokrows4, tokrows5, tokrows6, tokrows7)
    gt = (gt0, gt1, gt2, gt3, gt4, gt5, gt6, gt7)
    ot = (ot0, ot1, ot2, ot3, ot4, ot5, ot6, ot7)

    batch, lseq = tokens_hbm.shape
    blocks_total = batch * lseq // BLOCK
    blk_per_w = blocks_total // NW
    tok_per_w = blk_per_w * BLOCK
    n_ch = tok_per_w // CH
    npb = lseq // BLOCK
    spb = 2 * npb + lseq
    w_per_b = NW // batch

    wid = lax.axis_index("s") * NC + lax.axis_index("c")
    b = wid // w_per_b
    col0 = (wid - b * w_per_b) * tok_per_w

    pltpu.sync_copy(tokens_hbm.at[b, pl.ds(col0, tok_per_w)], tok_v)

    def issue(c, p):
        idx = tok_v.at[pl.ds(c * CH, CH)]
        pltpu.async_copy(token_w_hbm.at[idx], tokrows[p], gt[p])

    def wait_gather(p):
        pltpu.make_async_copy(token_w_hbm.at[pl.ds(0, CH)],
                              tokrows[p], gt[p]).wait()

    def drain_out(p):
        pltpu.make_async_copy(tokrows[p], seq_hbm.at[pl.ds(0, CH)],
                              ot[p]).wait()

    for c0 in range(LOOK):
        issue(c0, c0)

    def outer(kk, _):
        for i in range(RING):
            c = kk * RING + i
            nc = c + LOOK
            nbuf = (i + LOOK) % RING

            @pl.when(jnp.logical_and(c >= RING - LOOK, nc < n_ch))
            def _():
                drain_out(nbuf)

            @pl.when(nc < n_ch)
            def _():
                issue(nc, nbuf)

            wait_gather(i)
            row0 = b * spb + 2 * npb + col0 + c * CH
            pltpu.async_copy(tokrows[i], seq_hbm.at[pl.ds(row0, CH)], ot[i])
        return 0

    lax.fori_loop(0, n_ch // RING, outer, 0)
    for i in range(RING):
        drain_out(i)


def _tc_body(tokens_ref, cat_w_ref, num_w_ref,
             hist_ref, catnum_ref, cat_ids_ref):
    m = tokens_ref.shape[0]
    npb = 256
    batch = m // npb
    iota_v = lax.broadcasted_iota(jnp.int32, (m, VOCAB), 1)

    col0 = tokens_ref[:, 0:1]
    cat_ids_ref[...] = col0.reshape(batch, npb)

    h = jnp.zeros((m, VOCAB), jnp.float32)
    for r in range(BLOCK):
        col = tokens_ref[:, r:r + 1]
        h = h + (col == iota_v).astype(jnp.float32)
    hist_ref[...] = h

    onehot = (col0 == iota_v).astype(jnp.float32)
    cat = jnp.dot(onehot, cat_w_ref[...], preferred_element_type=jnp.float32)
    num = jnp.dot(h, num_w_ref[...], preferred_element_type=jnp.float32)
    for b in range(batch):
        catnum_ref[b * 2 * npb:b * 2 * npb + npb, :] = (
            cat[b * npb:(b + 1) * npb, :])
        catnum_ref[b * 2 * npb + npb:(b + 1) * 2 * npb, :] = (
            num[b * npb:(b + 1) * npb, :])


def kernel(tokens, cat_W, num_W, token_W):
    B, Lseq = tokens.shape
    n_blocks = Lseq // BLOCK
    seq_rows = 2 * n_blocks + Lseq

    mesh = plsc.VectorSubcoreMesh(core_axis_name="c", subcore_axis_name="s")
    sc = pl.kernel(
        _sc_body,
        out_type=jax.ShapeDtypeStruct((B * seq_rows, D), jnp.float32),
        mesh=mesh,
        compiler_params=pltpu.CompilerParams(needs_layout_passes=False),
        scratch_types=[
            pltpu.VMEM((Lseq * B // NW,), jnp.int32),
        ] + [pltpu.VMEM((CH, D), jnp.float32)] * RING
          + [pltpu.SemaphoreType.DMA] * (2 * RING),
    )

    tc = pl.pallas_call(
        _tc_body,
        out_shape=[
            jax.ShapeDtypeStruct((B * n_blocks, VOCAB), jnp.float32),
            jax.ShapeDtypeStruct((B * 2 * n_blocks, D), jnp.float32),
            jax.ShapeDtypeStruct((B, n_blocks), jnp.int32),
        ],
    )

    seq_flat = sc(tokens, token_W)
    hist_flat, catnum, cat_ids = tc(
        tokens.reshape(B * n_blocks, BLOCK), cat_W, num_W)

    new_seq = seq_flat.reshape(B, seq_rows, D)
    new_seq = lax.dynamic_update_slice(
        new_seq, catnum.reshape(B, 2 * n_blocks, D), (0, 0, 0))
    hist = hist_flat.reshape(B, n_blocks, VOCAB)
    return (new_seq, cat_ids, hist)
